# baseline (speedup 1.0000x reference)
import functools
import jax
import jax.numpy as jnp
from jax import lax
from jax.experimental import pallas as pl
from jax.experimental.pallas import tpu as pltpu
from jax.experimental.pallas import tpu_sc as plsc

_PMAX = 10.0
_B = 128
_C = 3
_H = 224
_W = 224
_N = _C * _H * _W
_ROWS = _C * _H
_L = 16
_VPR = _W // _L
_NW = 32
_SPW = _B // _NW
_CH_ROWS = 168
_CHUNK = _CH_ROWS * _W
_NCHUNK = _ROWS // _CH_ROWS
_NVEC = _CHUNK // _L
_U = 16
_NPAR = 6


def _floorv(y):
    t = y.astype(jnp.int32).astype(jnp.float32)
    return jnp.where(t > y, t - 1.0, t)


def _sc_body(x_hbm, par_hbm, samples_hbm, order_hbm, out_hbm,
             buf0, buf1, tmp, samp_v, par_v, ord_v, cnt,
             isem0, isem1, osem0, osem1):
    cid = lax.axis_index("c")
    sid = lax.axis_index("s")

    bufs = (buf0, buf1)
    isems = (isem0, isem1)
    osems = (osem0, osem1)

    pltpu.sync_copy(samples_hbm, samp_v)
    pltpu.sync_copy(par_hbm, par_v)
    pltpu.sync_copy(order_hbm, ord_v)

    bright_b = par_v[pl.ds(0 * _L, _L)]
    kc = par_v[pl.ds(1 * _L, _L)]
    thr = par_v[pl.ds(2 * _L, _L)]
    levels = par_v[pl.ds(3 * _L, _L)]
    inv_levels = par_v[pl.ds(4 * _L, _L)]
    ksh = par_v[pl.ds(5 * _L, _L)]
    ids = lax.iota(jnp.int32, 16)

    def chunk_sum(buf, acc0):
        z = jnp.zeros((_L,), jnp.float32)

        def vb(i, accs):
            a0, a1, a2, a3 = accs
            return (a0 + buf[pl.ds(i, _L)],
                    a1 + buf[pl.ds(i + _L, _L)],
                    a2 + buf[pl.ds(i + 2 * _L, _L)],
                    a3 + buf[pl.ds(i + 3 * _L, _L)])
        a0, a1, a2, a3 = plsc.parallel_loop(
            0, _CHUNK, step=4 * _L, unroll=4, carry=(acc0, z, z, z))(vb)
        return (a0 + a1) + (a2 + a3)

    def pointwise(buf, f):
        @plsc.parallel_loop(0, _CHUNK, step=_L, unroll=16)
        def _pw(i):
            sl = pl.ds(i, _L)
            buf[sl] = f(buf[sl])

    def tf_flip(buf):
        @plsc.parallel_loop(0, _CH_ROWS, step=1, unroll=2)
        def _fl(r):
            base = r * _W
            for j in range(_VPR // 2):
                lo = pl.ds(base + j * _L, _L)
                hi = pl.ds(base + (_VPR - 1 - j) * _L, _L)
                a = buf[lo]
                b = buf[hi]
                buf[lo] = lax.rev(b, (0,))
                buf[hi] = lax.rev(a, (0,))

    def tf_sharpness(buf):
        third = 1.0 / 3.0

        @plsc.parallel_loop(0, _CH_ROWS, step=1)
        def _sh(r):
            base = r * _W
            outs = []
            for j in range(_VPR):
                xv = buf[pl.ds(base + j * _L, _L)]
                if j == 0:
                    im = base + lax.rem(ids + (_W - 1), _W)
                    xm = plsc.load_gather(buf, [im])
                else:
                    xm = buf[pl.ds(base + j * _L - 1, _L)]
                if j == _VPR - 1:
                    ip = base + lax.rem(ids + (j * _L + 1), _W)
                    xp = plsc.load_gather(buf, [ip])
                else:
                    xp = buf[pl.ds(base + j * _L + 1, _L)]
                blur = (xv + xm + xp) * third
                outs.append(xv + ksh * (xv - blur))
            for j in range(_VPR):
                buf[pl.ds(base + j * _L, _L)] = outs[j]

    def apply_round(buf, tf, mean_scalar, do_contrast):
        def c_contrast():
            if do_contrast:
                pointwise(buf, lambda v: (v - mean_scalar) * kc + mean_scalar)
        lax.cond(
            tf < 4,
            lambda: lax.cond(
                tf < 2,
                lambda: lax.cond(
                    tf == 0,
                    lambda: None,
                    lambda: pointwise(
                        buf, lambda v: jnp.clip(v + bright_b, 0.0, 1.0))),
                lambda: lax.cond(
                    tf == 2,
                    c_contrast,
                    lambda: pointwise(buf, lambda v: 1.0 - v))),
            lambda: lax.cond(
                tf < 6,
                lambda: lax.cond(
                    tf == 4,
                    lambda: pointwise(
                        buf, lambda v: jnp.where(v < thr, v, 1.0 - v)),
                    lambda: pointwise(
                        buf,
                        lambda v: _floorv(v * levels) * inv_levels)),
                lambda: lax.cond(
                    tf == 6,
                    lambda: tf_flip(buf),
                    lambda: tf_sharpness(buf))))

    zero16 = jnp.zeros((_L,), jnp.float32)
    inv_n = 1.0 / float(_N)

    def process_sample(s):
        sidx = jnp.full((_L,), s, jnp.int32)
        tf0 = plsc.load_gather(samp_v, [sidx])[0]
        tf1 = plsc.load_gather(samp_v, [sidx + _B])[0]
        tf0_contrast = tf0 == 2
        tf1_contrast = tf1 == 2

        def mean_in():
            def cb(c, acc):
                pltpu.sync_copy(x_hbm.at[s, pl.ds(c * _CHUNK, _CHUNK)], buf0)
                return chunk_sum(buf0, acc)
            acc = lax.fori_loop(0, _NCHUNK, cb, zero16)
            return jnp.sum(acc) * inv_n
        m0 = lax.cond(tf0_contrast, mean_in, lambda: 0.0)

        cin = [pltpu.make_async_copy(
                   x_hbm.at[s, pl.ds(c * _CHUNK, _CHUNK)],
                   bufs[c % 2], isems[c % 2]) for c in range(_NCHUNK)]
        cout = [pltpu.make_async_copy(
                    bufs[c % 2],
                    out_hbm.at[s, pl.ds(c * _CHUNK, _CHUNK)],
                    osems[c % 2]) for c in range(_NCHUNK)]

        cin[0].start()
        sum1 = zero16
        for c in range(_NCHUNK):
            if c + 1 < _NCHUNK:
                if c >= 1:
                    cout[c - 1].wait()
                cin[c + 1].start()
            cin[c].wait()
            b = bufs[c % 2]
            apply_round(b, tf0, m0, True)
            sum1 = lax.cond(tf1_contrast,
                            functools.partial(chunk_sum, b, sum1),
                            lambda: sum1)
            apply_round(b, tf1, 0.0, False)
            cout[c].start()
        cout[_NCHUNK - 2].wait()
        cout[_NCHUNK - 1].wait()

        def fix_contrast():
            m1 = jnp.sum(sum1) * inv_n
            def cb3(c, carry2):
                pltpu.sync_copy(out_hbm.at[s, pl.ds(c * _CHUNK, _CHUNK)], buf0)
                pointwise(buf0, lambda v: (v - m1) * kc + m1)
                pltpu.sync_copy(buf0, out_hbm.at[s, pl.ds(c * _CHUNK, _CHUNK)])
                return carry2
            lax.fori_loop(0, _NCHUNK, cb3, 0)
        pl.when(tf1_contrast)(fix_contrast)

    half = _B // 2
    def _init_counter():
        cnt[0] = 0
    pl.when(sid == 0)(_init_counter)
    plsc.subcore_barrier()

    def w_cond(t):
        return t < half

    def w_body(t):
        pos = jnp.full((_L,), cid * half + t, jnp.int32)
        process_sample(plsc.load_gather(ord_v, [pos])[0])
        return plsc.fetch_and_add(cnt.at[0], 1, subcore_id=0)

    t0 = plsc.fetch_and_add(cnt.at[0], 1, subcore_id=0)
    lax.while_loop(w_cond, w_body, t0)


def kernel(x, mag, samples):
    x2 = x.reshape(_B, _N)
    m = jnp.asarray(mag, jnp.float32).reshape(())
    magd = m / _PMAX
    levels = 2.0 + jnp.floor(m)
    par = jnp.stack([
        magd - 0.5,
        0.5 + magd,
        magd,
        levels,
        1.0 / levels,
        magd,
    ])
    par16 = jnp.repeat(par, _L)
    samp = samples.astype(jnp.int32).reshape(2 * _B)

    w0 = jnp.array([0.0, 1.0, 2.3, 1.0, 1.0, 1.6, 0.8, 3.0], jnp.float32)
    w1 = jnp.array([0.0, 1.0, 3.4, 1.0, 1.0, 1.6, 0.8, 3.0], jnp.float32)
    cost = w0[samp[:_B]] + w1[samp[_B:]]
    order_sorted = jnp.argsort(-cost).astype(jnp.int32)
    order = jnp.concatenate([order_sorted[0::2], order_sorted[1::2]])

    fn = pl.kernel(
        _sc_body,
        out_type=jax.ShapeDtypeStruct((_B, _N), jnp.float32),
        mesh=plsc.VectorSubcoreMesh(core_axis_name="c", subcore_axis_name="s"),
        scratch_types=[
            pltpu.VMEM((_CHUNK,), jnp.float32),
            pltpu.VMEM((_CHUNK,), jnp.float32),
            pltpu.VMEM((_W,), jnp.float32),
            pltpu.VMEM((2 * _B,), jnp.int32),
            pltpu.VMEM((_NPAR * _L,), jnp.float32),
            pltpu.VMEM((_B,), jnp.int32),
            pltpu.SMEM((1,), jnp.int32),
            pltpu.SemaphoreType.DMA,
            pltpu.SemaphoreType.DMA,
            pltpu.SemaphoreType.DMA,
            pltpu.SemaphoreType.DMA,
        ],
        compiler_params=pltpu.CompilerParams(needs_layout_passes=False),
    )
    out = fn(x2, par16, samp, order)
    return out.reshape(_B, _C, _H, _W)


# speedup vs baseline: 1.0072x; 1.0072x over previous
import functools
import jax
import jax.numpy as jnp
from jax import lax
from jax.experimental import pallas as pl
from jax.experimental.pallas import tpu as pltpu
from jax.experimental.pallas import tpu_sc as plsc

_PMAX = 10.0
_B = 128
_C = 3
_H = 224
_W = 224
_N = _C * _H * _W
_ROWS = _C * _H
_L = 16
_VPR = _W // _L
_NW = 32
_SPW = _B // _NW
_CH_ROWS = 168
_CHUNK = _CH_ROWS * _W
_NCHUNK = _ROWS // _CH_ROWS
_NVEC = _CHUNK // _L
_U = 16
_NPAR = 6


def _floorv(y):
    t = y.astype(jnp.int32).astype(jnp.float32)
    return jnp.where(t > y, t - 1.0, t)


def _sc_body(x_hbm, par_hbm, samples_hbm, order_hbm, out_hbm,
             buf0, buf1, tmp, samp_v, par_v, ord_v, cnt,
             isem0, isem1, osem0, osem1):
    cid = lax.axis_index("c")
    sid = lax.axis_index("s")

    bufs = (buf0, buf1)
    isems = (isem0, isem1)
    osems = (osem0, osem1)

    pltpu.sync_copy(samples_hbm, samp_v)
    pltpu.sync_copy(par_hbm, par_v)
    pltpu.sync_copy(order_hbm, ord_v)

    bright_b = par_v[pl.ds(0 * _L, _L)]
    kc = par_v[pl.ds(1 * _L, _L)]
    thr = par_v[pl.ds(2 * _L, _L)]
    levels = par_v[pl.ds(3 * _L, _L)]
    inv_levels = par_v[pl.ds(4 * _L, _L)]
    ksh = par_v[pl.ds(5 * _L, _L)]
    ids = lax.iota(jnp.int32, 16)

    def chunk_sum(buf, acc0):
        z = jnp.zeros((_L,), jnp.float32)

        def vb(i, accs):
            a0, a1, a2, a3 = accs
            return (a0 + buf[pl.ds(i, _L)],
                    a1 + buf[pl.ds(i + _L, _L)],
                    a2 + buf[pl.ds(i + 2 * _L, _L)],
                    a3 + buf[pl.ds(i + 3 * _L, _L)])
        a0, a1, a2, a3 = plsc.parallel_loop(
            0, _CHUNK, step=4 * _L, unroll=4, carry=(acc0, z, z, z))(vb)
        return (a0 + a1) + (a2 + a3)

    def pointwise(buf, f):
        @plsc.parallel_loop(0, _CHUNK, step=_L, unroll=8)
        def _pw(i):
            sl = pl.ds(i, _L)
            buf[sl] = f(buf[sl])

    def tf_flip(buf):
        @plsc.parallel_loop(0, _CH_ROWS, step=1, unroll=2)
        def _fl(r):
            base = r * _W
            for j in range(_VPR // 2):
                lo = pl.ds(base + j * _L, _L)
                hi = pl.ds(base + (_VPR - 1 - j) * _L, _L)
                a = buf[lo]
                b = buf[hi]
                buf[lo] = lax.rev(b, (0,))
                buf[hi] = lax.rev(a, (0,))

    def tf_sharpness(buf):
        third = 1.0 / 3.0

        @plsc.parallel_loop(0, _CH_ROWS, step=1)
        def _sh(r):
            base = r * _W
            outs = []
            for j in range(_VPR):
                xv = buf[pl.ds(base + j * _L, _L)]
                if j == 0:
                    im = base + lax.rem(ids + (_W - 1), _W)
                    xm = plsc.load_gather(buf, [im])
                else:
                    xm = buf[pl.ds(base + j * _L - 1, _L)]
                if j == _VPR - 1:
                    ip = base + lax.rem(ids + (j * _L + 1), _W)
                    xp = plsc.load_gather(buf, [ip])
                else:
                    xp = buf[pl.ds(base + j * _L + 1, _L)]
                blur = (xv + xm + xp) * third
                outs.append(xv + ksh * (xv - blur))
            for j in range(_VPR):
                buf[pl.ds(base + j * _L, _L)] = outs[j]

    def apply_round(buf, tf, mean_scalar, do_contrast):
        def c_contrast():
            if do_contrast:
                pointwise(buf, lambda v: (v - mean_scalar) * kc + mean_scalar)
        lax.cond(
            tf < 4,
            lambda: lax.cond(
                tf < 2,
                lambda: lax.cond(
                    tf == 0,
                    lambda: None,
                    lambda: pointwise(
                        buf, lambda v: jnp.clip(v + bright_b, 0.0, 1.0))),
                lambda: lax.cond(
                    tf == 2,
                    c_contrast,
                    lambda: pointwise(buf, lambda v: 1.0 - v))),
            lambda: lax.cond(
                tf < 6,
                lambda: lax.cond(
                    tf == 4,
                    lambda: pointwise(
                        buf, lambda v: jnp.where(v < thr, v, 1.0 - v)),
                    lambda: pointwise(
                        buf,
                        lambda v: _floorv(v * levels) * inv_levels)),
                lambda: lax.cond(
                    tf == 6,
                    lambda: tf_flip(buf),
                    lambda: tf_sharpness(buf))))

    zero16 = jnp.zeros((_L,), jnp.float32)
    inv_n = 1.0 / float(_N)

    def process_sample(s):
        sidx = jnp.full((_L,), s, jnp.int32)
        tf0 = plsc.load_gather(samp_v, [sidx])[0]
        tf1 = plsc.load_gather(samp_v, [sidx + _B])[0]
        tf0_contrast = tf0 == 2
        tf1_contrast = tf1 == 2

        def mean_in():
            def cb(c, acc):
                pltpu.sync_copy(x_hbm.at[s, pl.ds(c * _CHUNK, _CHUNK)], buf0)
                return chunk_sum(buf0, acc)
            acc = lax.fori_loop(0, _NCHUNK, cb, zero16)
            return jnp.sum(acc) * inv_n
        m0 = lax.cond(tf0_contrast, mean_in, lambda: 0.0)

        cin = [pltpu.make_async_copy(
                   x_hbm.at[s, pl.ds(c * _CHUNK, _CHUNK)],
                   bufs[c % 2], isems[c % 2]) for c in range(_NCHUNK)]
        cout = [pltpu.make_async_copy(
                    bufs[c % 2],
                    out_hbm.at[s, pl.ds(c * _CHUNK, _CHUNK)],
                    osems[c % 2]) for c in range(_NCHUNK)]

        cin[0].start()
        sum1 = zero16
        for c in range(_NCHUNK):
            if c + 1 < _NCHUNK:
                if c >= 1:
                    cout[c - 1].wait()
                cin[c + 1].start()
            cin[c].wait()
            b = bufs[c % 2]
            apply_round(b, tf0, m0, True)
            sum1 = lax.cond(tf1_contrast,
                            functools.partial(chunk_sum, b, sum1),
                            lambda: sum1)
            apply_round(b, tf1, 0.0, False)
            cout[c].start()
        cout[_NCHUNK - 2].wait()
        cout[_NCHUNK - 1].wait()

        def fix_contrast():
            m1 = jnp.sum(sum1) * inv_n
            def cb3(c, carry2):
                pltpu.sync_copy(out_hbm.at[s, pl.ds(c * _CHUNK, _CHUNK)], buf0)
                pointwise(buf0, lambda v: (v - m1) * kc + m1)
                pltpu.sync_copy(buf0, out_hbm.at[s, pl.ds(c * _CHUNK, _CHUNK)])
                return carry2
            lax.fori_loop(0, _NCHUNK, cb3, 0)
        pl.when(tf1_contrast)(fix_contrast)

    half = _B // 2
    def _init_counter():
        cnt[0] = 0
    pl.when(sid == 0)(_init_counter)
    plsc.subcore_barrier()

    def w_cond(t):
        return t < half

    def w_body(t):
        pos = jnp.full((_L,), cid * half + t, jnp.int32)
        process_sample(plsc.load_gather(ord_v, [pos])[0])
        return plsc.fetch_and_add(cnt.at[0], 1, subcore_id=0)

    t0 = plsc.fetch_and_add(cnt.at[0], 1, subcore_id=0)
    lax.while_loop(w_cond, w_body, t0)


def kernel(x, mag, samples):
    x2 = x.reshape(_B, _N)
    m = jnp.asarray(mag, jnp.float32).reshape(())
    magd = m / _PMAX
    levels = 2.0 + jnp.floor(m)
    par = jnp.stack([
        magd - 0.5,
        0.5 + magd,
        magd,
        levels,
        1.0 / levels,
        magd,
    ])
    par16 = jnp.repeat(par, _L)
    samp = samples.astype(jnp.int32).reshape(2 * _B)

    w0 = jnp.array([0.0, 1.0, 2.3, 1.0, 1.0, 1.6, 0.8, 3.0], jnp.float32)
    w1 = jnp.array([0.0, 1.0, 3.4, 1.0, 1.0, 1.6, 0.8, 3.0], jnp.float32)
    cost = w0[samp[:_B]] + w1[samp[_B:]]
    order_sorted = jnp.argsort(-cost).astype(jnp.int32)
    order = jnp.concatenate([order_sorted[0::2], order_sorted[1::2]])

    fn = pl.kernel(
        _sc_body,
        out_type=jax.ShapeDtypeStruct((_B, _N), jnp.float32),
        mesh=plsc.VectorSubcoreMesh(core_axis_name="c", subcore_axis_name="s"),
        scratch_types=[
            pltpu.VMEM((_CHUNK,), jnp.float32),
            pltpu.VMEM((_CHUNK,), jnp.float32),
            pltpu.VMEM((_W,), jnp.float32),
            pltpu.VMEM((2 * _B,), jnp.int32),
            pltpu.VMEM((_NPAR * _L,), jnp.float32),
            pltpu.VMEM((_B,), jnp.int32),
            pltpu.SMEM((1,), jnp.int32),
            pltpu.SemaphoreType.DMA,
            pltpu.SemaphoreType.DMA,
            pltpu.SemaphoreType.DMA,
            pltpu.SemaphoreType.DMA,
        ],
        compiler_params=pltpu.CompilerParams(needs_layout_passes=False),
    )
    out = fn(x2, par16, samp, order)
    return out.reshape(_B, _C, _H, _W)
